# baseline (device time: 105438 ns/iter reference)
import jax
import jax.numpy as jnp
from jax import lax
from jax.experimental import pallas as pl
from jax.experimental.pallas import tpu as pltpu

N_DEV = 16
W_BUFS = 3
W_SUBS = 4
B_F32 = 8


def kernel(x, w_mat):
    m_loc, k = x.shape
    _, n = w_mat.shape
    n_loc = n // N_DEV
    rows = k // W_SUBS
    f8 = jnp.float8_e4m3fn
    n_e4 = N_DEV - 1 - B_F32

    def body(x_ref, w_hbm, out_ref, w_buf, y_late, q_send, q_recv, maxes,
             w_sems, f_send_sems, f_recv_sems, q_send_sems, q_recv_sems,
             max_send_sems, max_recv_sems):
        my = lax.axis_index("i")

        def f32_rdma(d, src_ref, dev):
            return pltpu.make_async_remote_copy(
                src_ref=src_ref,
                dst_ref=out_ref.at[pl.ds(my * m_loc, m_loc), :],
                send_sem=f_send_sems.at[d - 1],
                recv_sem=f_recv_sems.at[d - 1],
                device_id=(dev,),
                device_id_type=pl.DeviceIdType.MESH)

        def f32_rdma_recv(d):
            s = (my - d) % N_DEV
            return pltpu.make_async_remote_copy(
                src_ref=y_late.at[0],
                dst_ref=out_ref.at[pl.ds(s * m_loc, m_loc), :],
                send_sem=f_send_sems.at[d - 1],
                recv_sem=f_recv_sems.at[d - 1],
                device_id=(my,),
                device_id_type=pl.DeviceIdType.MESH)

        def e4_rdma(d, dev):
            return pltpu.make_async_remote_copy(
                src_ref=q_send.at[d - B_F32 - 1],
                dst_ref=q_recv.at[d - B_F32 - 1],
                send_sem=q_send_sems.at[d - B_F32 - 1],
                recv_sem=q_recv_sems.at[d - B_F32 - 1],
                device_id=(dev,),
                device_id_type=pl.DeviceIdType.MESH)

        def max_rdma(d, dev):
            return pltpu.make_async_remote_copy(
                src_ref=maxes.at[0], dst_ref=maxes.at[d],
                send_sem=max_send_sems.at[d],
                recv_sem=max_recv_sems.at[d],
                device_id=(dev,),
                device_id_type=pl.DeviceIdType.MESH)

        barrier = pltpu.get_barrier_semaphore()
        for d in range(1, N_DEV):
            peer = (my + d) % N_DEV
            pl.semaphore_signal(barrier, inc=1, device_id=(peer,),
                                device_id_type=pl.DeviceIdType.MESH)
        pl.semaphore_wait(barrier, N_DEV - 1)

        order = list(range(1, N_DEV)) + [0]

        def start_w(idx, slot):
            t = (my + order[idx]) % N_DEV
            for j in range(W_SUBS):
                pltpu.make_async_copy(
                    w_hbm.at[pl.ds(j * rows, rows), pl.ds(t * n_loc, n_loc)],
                    w_buf.at[slot, pl.ds(j * rows, rows), :],
                    w_sems.at[slot, j]).start()

        def wait_w(slot):
            for j in range(W_SUBS):
                pltpu.make_async_copy(
                    w_hbm.at[pl.ds(0, rows), pl.ds(0, n_loc)],
                    w_buf.at[slot, pl.ds(j * rows, rows), :],
                    w_sems.at[slot, j]).wait()

        for p in range(W_BUFS - 1):
            start_w(p, p)
        m_acc = jnp.float32(0.0)
        for idx, d in enumerate(order):
            slot = idx % W_BUFS
            wait_w(slot)
            nxt = idx + W_BUFS - 1
            if nxt < N_DEV:
                start_w(nxt, nxt % W_BUFS)
            y = jnp.maximum(
                jnp.dot(x_ref[:], w_buf[slot],
                        preferred_element_type=jnp.float32),
                0.0)
            m_acc = jnp.maximum(m_acc, jnp.max(y))
            t = (my + d) % N_DEV
            if 1 <= d < B_F32:
                out_ref[pl.ds(t * m_loc, m_loc), :] = y
                f32_rdma(d, out_ref.at[pl.ds(t * m_loc, m_loc), :], t).start()
            elif d == B_F32:
                y_late[0] = y
                f32_rdma(d, y_late.at[0], t).start()
            elif d > B_F32:
                y_late[d - B_F32] = y
            else:
                out_ref[pl.ds(my * m_loc, m_loc), :] = y

        maxes[0] = jnp.full((1, 128), m_acc, jnp.float32)
        for d in range(1, N_DEV):
            max_rdma(d, (my + d) % N_DEV).start()
        for d in range(1, N_DEV):
            max_rdma(d, my).wait_recv()

        g = jnp.max(maxes[...])
        inv = 448.0 / g
        scale = g / 448.0

        for d in range(B_F32 + 1, N_DEV):
            t = (my + d) % N_DEV
            q_send[d - B_F32 - 1] = (y_late[d - B_F32] * inv).astype(f8)
            e4_rdma(d, t).start()

        out_ref[pl.ds(my * m_loc, m_loc), :] = (
            (out_ref[pl.ds(my * m_loc, m_loc), :] * inv).astype(f8)
            .astype(jnp.float32) * scale)

        for d in range(1, B_F32):
            t = (my + d) % N_DEV
            f32_rdma(d, out_ref.at[pl.ds(t * m_loc, m_loc), :],
                     t).wait_send()

        for d in range(1, B_F32 + 1):
            s = (my - d) % N_DEV
            f32_rdma_recv(d).wait_recv()
            out_ref[pl.ds(s * m_loc, m_loc), :] = (
                (out_ref[pl.ds(s * m_loc, m_loc), :] * inv).astype(f8)
                .astype(jnp.float32) * scale)

        for d in range(B_F32 + 1, N_DEV):
            s = (my - d) % N_DEV
            e4_rdma(d, my).wait_recv()
            out_ref[pl.ds(s * m_loc, m_loc), :] = (
                q_recv[d - B_F32 - 1].astype(jnp.float32) * scale)

        f32_rdma(B_F32, y_late.at[0], (my + B_F32) % N_DEV).wait_send()
        for d in range(B_F32 + 1, N_DEV):
            e4_rdma(d, (my + d) % N_DEV).wait_send()
        for d in range(1, N_DEV):
            max_rdma(d, (my + d) % N_DEV).wait_send()

    return pl.pallas_call(
        body,
        out_shape=jax.ShapeDtypeStruct((m_loc * N_DEV, n_loc), jnp.float32),
        in_specs=[
            pl.BlockSpec(memory_space=pltpu.VMEM),
            pl.BlockSpec(memory_space=pl.ANY),
        ],
        out_specs=pl.BlockSpec(memory_space=pltpu.VMEM),
        scratch_shapes=[
            pltpu.VMEM((W_BUFS, k, n_loc), jnp.float32),
            pltpu.VMEM((N_DEV - B_F32, m_loc, n_loc), jnp.float32),
            pltpu.VMEM((n_e4, m_loc, n_loc), f8),
            pltpu.VMEM((n_e4, m_loc, n_loc), f8),
            pltpu.VMEM((N_DEV, 1, 128), jnp.float32),
            pltpu.SemaphoreType.DMA((W_BUFS, W_SUBS)),
            pltpu.SemaphoreType.DMA((B_F32,)),
            pltpu.SemaphoreType.DMA((B_F32,)),
            pltpu.SemaphoreType.DMA((n_e4,)),
            pltpu.SemaphoreType.DMA((n_e4,)),
            pltpu.SemaphoreType.DMA((N_DEV,)),
            pltpu.SemaphoreType.DMA((N_DEV,)),
        ],
        compiler_params=pltpu.CompilerParams(collective_id=0),
    )(x, w_mat)


# device time: 80197 ns/iter; 1.3147x vs baseline; 1.3147x over previous
import jax
import jax.numpy as jnp
from jax import lax
from jax.experimental import pallas as pl
from jax.experimental.pallas import tpu as pltpu

N_DEV = 16
W_BUFS = 3
W_SUBS = 4


def kernel(x, w_mat):
    m_loc, k = x.shape
    _, n = w_mat.shape
    n_loc = n // N_DEV
    rows = k // W_SUBS
    f8 = jnp.float8_e4m3fn

    def body(x_ref, w_hbm, out_ref, w_buf, q_send, q_recv, maxes,
             w_sems, q_send_sems, q_recv_sems, max_send_sems, max_recv_sems):
        my = lax.axis_index("i")

        barrier = pltpu.get_barrier_semaphore()
        for d in range(1, N_DEV):
            peer = (my + d) % N_DEV
            pl.semaphore_signal(barrier, inc=1, device_id=(peer,),
                                device_id_type=pl.DeviceIdType.MESH)
        pl.semaphore_wait(barrier, N_DEV - 1)

        order = list(range(1, N_DEV)) + [0]

        def start_w(idx, slot):
            t = (my + order[idx]) % N_DEV
            for j in range(W_SUBS):
                pltpu.make_async_copy(
                    w_hbm.at[pl.ds(j * rows, rows), pl.ds(t * n_loc, n_loc)],
                    w_buf.at[slot, pl.ds(j * rows, rows), :],
                    w_sems.at[slot, j]).start()

        def wait_w_sub(slot, j):
            pltpu.make_async_copy(
                w_hbm.at[pl.ds(0, rows), pl.ds(0, n_loc)],
                w_buf.at[slot, pl.ds(j * rows, rows), :],
                w_sems.at[slot, j]).wait()

        for p in range(W_BUFS - 1):
            start_w(p, p)
        m_acc = jnp.float32(0.0)
        for idx, d in enumerate(order):
            slot = idx % W_BUFS
            nxt = idx + W_BUFS - 1
            if nxt < N_DEV:
                start_w(nxt, nxt % W_BUFS)
            y = None
            for j in range(W_SUBS):
                wait_w_sub(slot, j)
                p = jnp.dot(x_ref[:, pl.ds(j * rows, rows)],
                            w_buf[slot, pl.ds(j * rows, rows), :],
                            preferred_element_type=jnp.float32)
                y = p if y is None else y + p
            y = jnp.maximum(y, 0.0)
            m_acc = jnp.maximum(m_acc, jnp.max(y))
            t = (my + d) % N_DEV
            out_ref[pl.ds(t * m_loc, m_loc), :] = y

        maxes[0] = jnp.full((1, 128), m_acc, jnp.float32)
        for d in range(1, N_DEV):
            t = (my + d) % N_DEV
            pltpu.make_async_remote_copy(
                src_ref=maxes.at[0], dst_ref=maxes.at[d],
                send_sem=max_send_sems.at[d],
                recv_sem=max_recv_sems.at[d],
                device_id=(t,),
                device_id_type=pl.DeviceIdType.MESH).start()
        for d in range(1, N_DEV):
            pltpu.make_async_remote_copy(
                src_ref=maxes.at[0], dst_ref=maxes.at[d],
                send_sem=max_send_sems.at[d],
                recv_sem=max_recv_sems.at[d],
                device_id=(my,),
                device_id_type=pl.DeviceIdType.MESH).wait_recv()

        g = jnp.max(maxes[...])
        inv = 448.0 / g
        scale = g / 448.0

        for d in range(1, N_DEV):
            t = (my + d) % N_DEV
            q_send[d] = (out_ref[pl.ds(t * m_loc, m_loc), :] * inv).astype(f8)
            pltpu.make_async_remote_copy(
                src_ref=q_send.at[d],
                dst_ref=q_recv.at[d],
                send_sem=q_send_sems.at[d],
                recv_sem=q_recv_sems.at[d],
                device_id=(t,),
                device_id_type=pl.DeviceIdType.MESH).start()
        out_ref[pl.ds(my * m_loc, m_loc), :] = (
            (out_ref[pl.ds(my * m_loc, m_loc), :] * inv).astype(f8)
            .astype(jnp.float32) * scale)

        for d in range(1, N_DEV):
            s = (my - d) % N_DEV
            pltpu.make_async_remote_copy(
                src_ref=q_send.at[d],
                dst_ref=q_recv.at[d],
                send_sem=q_send_sems.at[d],
                recv_sem=q_recv_sems.at[d],
                device_id=(my,),
                device_id_type=pl.DeviceIdType.MESH).wait_recv()
            out_ref[pl.ds(s * m_loc, m_loc), :] = (
                q_recv[d].astype(jnp.float32) * scale)

        for d in range(1, N_DEV):
            t = (my + d) % N_DEV
            pltpu.make_async_remote_copy(
                src_ref=q_send.at[d],
                dst_ref=q_recv.at[d],
                send_sem=q_send_sems.at[d],
                recv_sem=q_recv_sems.at[d],
                device_id=(t,),
                device_id_type=pl.DeviceIdType.MESH).wait_send()
            pltpu.make_async_remote_copy(
                src_ref=maxes.at[0], dst_ref=maxes.at[d],
                send_sem=max_send_sems.at[d],
                recv_sem=max_recv_sems.at[d],
                device_id=(t,),
                device_id_type=pl.DeviceIdType.MESH).wait_send()

    return pl.pallas_call(
        body,
        out_shape=jax.ShapeDtypeStruct((m_loc * N_DEV, n_loc), jnp.float32),
        in_specs=[
            pl.BlockSpec(memory_space=pltpu.VMEM),
            pl.BlockSpec(memory_space=pl.ANY),
        ],
        out_specs=pl.BlockSpec(memory_space=pltpu.VMEM),
        scratch_shapes=[
            pltpu.VMEM((W_BUFS, k, n_loc), jnp.float32),
            pltpu.VMEM((N_DEV, m_loc, n_loc), f8),
            pltpu.VMEM((N_DEV, m_loc, n_loc), f8),
            pltpu.VMEM((N_DEV, 1, 128), jnp.float32),
            pltpu.SemaphoreType.DMA((W_BUFS, W_SUBS)),
            pltpu.SemaphoreType.DMA((N_DEV,)),
            pltpu.SemaphoreType.DMA((N_DEV,)),
            pltpu.SemaphoreType.DMA((N_DEV,)),
            pltpu.SemaphoreType.DMA((N_DEV,)),
        ],
        compiler_params=pltpu.CompilerParams(collective_id=0),
    )(x, w_mat)


# device time: 79441 ns/iter; 1.3272x vs baseline; 1.0095x over previous
import jax
import jax.numpy as jnp
from jax import lax
from jax.experimental import pallas as pl
from jax.experimental.pallas import tpu as pltpu

N_DEV = 16
W_BUFS = 3
W_SUBS = 4


def kernel(x, w_mat):
    m_loc, k = x.shape
    _, n = w_mat.shape
    n_loc = n // N_DEV
    rows = k // W_SUBS
    f8 = jnp.float8_e4m3fn

    def body(x_ref, w_hbm, out_ref, w_buf, q_send, q_recv, maxes,
             w_sems, q_send_sems, q_recv_sems, max_send_sems, max_recv_sems):
        my = lax.axis_index("i")

        barrier = pltpu.get_barrier_semaphore()
        for d in range(1, N_DEV):
            peer = (my + d) % N_DEV
            pl.semaphore_signal(barrier, inc=1, device_id=(peer,),
                                device_id_type=pl.DeviceIdType.MESH)
        pl.semaphore_wait(barrier, N_DEV - 1)

        order = list(range(1, N_DEV)) + [0]

        def start_w(idx, slot):
            t = (my + order[idx]) % N_DEV
            for j in range(W_SUBS):
                pltpu.make_async_copy(
                    w_hbm.at[pl.ds(j * rows, rows), pl.ds(t * n_loc, n_loc)],
                    w_buf.at[slot, pl.ds(j * rows, rows), :],
                    w_sems.at[slot, j]).start()

        def wait_w(slot):
            for j in range(W_SUBS):
                pltpu.make_async_copy(
                    w_hbm.at[pl.ds(0, rows), pl.ds(0, n_loc)],
                    w_buf.at[slot, pl.ds(j * rows, rows), :],
                    w_sems.at[slot, j]).wait()

        for p in range(W_BUFS - 1):
            start_w(p, p)
        m_acc = jnp.float32(0.0)
        for idx, d in enumerate(order):
            slot = idx % W_BUFS
            wait_w(slot)
            nxt = idx + W_BUFS - 1
            if nxt < N_DEV:
                start_w(nxt, nxt % W_BUFS)
            y = jnp.maximum(
                jnp.dot(x_ref[:], w_buf[slot],
                        preferred_element_type=jnp.float32),
                0.0)
            m_acc = jnp.maximum(m_acc, jnp.max(y))
            t = (my + d) % N_DEV
            out_ref[pl.ds(t * m_loc, m_loc), :] = y

        maxes[0] = jnp.full((1, 128), m_acc, jnp.float32)
        for d in range(1, N_DEV):
            t = (my + d) % N_DEV
            pltpu.make_async_remote_copy(
                src_ref=maxes.at[0], dst_ref=maxes.at[d],
                send_sem=max_send_sems.at[d],
                recv_sem=max_recv_sems.at[d],
                device_id=(t,),
                device_id_type=pl.DeviceIdType.MESH).start()
        for d in range(1, N_DEV):
            pltpu.make_async_remote_copy(
                src_ref=maxes.at[0], dst_ref=maxes.at[d],
                send_sem=max_send_sems.at[d],
                recv_sem=max_recv_sems.at[d],
                device_id=(my,),
                device_id_type=pl.DeviceIdType.MESH).wait_recv()

        g = jnp.max(maxes[...])
        inv = 448.0 / g
        scale = g / 448.0

        for d in range(1, N_DEV):
            t = (my + d) % N_DEV
            q_send[d] = (out_ref[pl.ds(t * m_loc, m_loc), :] * inv).astype(f8)
            pltpu.make_async_remote_copy(
                src_ref=q_send.at[d],
                dst_ref=q_recv.at[d],
                send_sem=q_send_sems.at[d],
                recv_sem=q_recv_sems.at[d],
                device_id=(t,),
                device_id_type=pl.DeviceIdType.MESH).start()
        out_ref[pl.ds(my * m_loc, m_loc), :] = (
            (out_ref[pl.ds(my * m_loc, m_loc), :] * inv).astype(f8)
            .astype(jnp.float32) * scale)

        for d in range(1, N_DEV):
            s = (my - d) % N_DEV
            pltpu.make_async_remote_copy(
                src_ref=q_send.at[d],
                dst_ref=q_recv.at[d],
                send_sem=q_send_sems.at[d],
                recv_sem=q_recv_sems.at[d],
                device_id=(my,),
                device_id_type=pl.DeviceIdType.MESH).wait_recv()
            out_ref[pl.ds(s * m_loc, m_loc), :] = (
                q_recv[d].astype(jnp.float32) * scale)

        for d in range(1, N_DEV):
            t = (my + d) % N_DEV
            pltpu.make_async_remote_copy(
                src_ref=q_send.at[d],
                dst_ref=q_recv.at[d],
                send_sem=q_send_sems.at[d],
                recv_sem=q_recv_sems.at[d],
                device_id=(t,),
                device_id_type=pl.DeviceIdType.MESH).wait_send()
            pltpu.make_async_remote_copy(
                src_ref=maxes.at[0], dst_ref=maxes.at[d],
                send_sem=max_send_sems.at[d],
                recv_sem=max_recv_sems.at[d],
                device_id=(t,),
                device_id_type=pl.DeviceIdType.MESH).wait_send()

    return pl.pallas_call(
        body,
        out_shape=jax.ShapeDtypeStruct((m_loc * N_DEV, n_loc), jnp.float32),
        in_specs=[
            pl.BlockSpec(memory_space=pltpu.VMEM),
            pl.BlockSpec(memory_space=pl.ANY),
        ],
        out_specs=pl.BlockSpec(memory_space=pltpu.VMEM),
        scratch_shapes=[
            pltpu.VMEM((W_BUFS, k, n_loc), jnp.float32),
            pltpu.VMEM((N_DEV, m_loc, n_loc), f8),
            pltpu.VMEM((N_DEV, m_loc, n_loc), f8),
            pltpu.VMEM((N_DEV, 1, 128), jnp.float32),
            pltpu.SemaphoreType.DMA((W_BUFS, W_SUBS)),
            pltpu.SemaphoreType.DMA((N_DEV,)),
            pltpu.SemaphoreType.DMA((N_DEV,)),
            pltpu.SemaphoreType.DMA((N_DEV,)),
            pltpu.SemaphoreType.DMA((N_DEV,)),
        ],
        compiler_params=pltpu.CompilerParams(collective_id=0),
    )(x, w_mat)
